# baseline (speedup 1.0000x reference)
import functools

import jax
import jax.numpy as jnp
from jax import lax
from jax.experimental import pallas as pl
from jax.experimental.pallas import tpu as pltpu
from jax.experimental.pallas import tpu_sc as plsc

NC, NS, L = 2, 16, 16
NW = NC * NS
B, C, H, W = 8, 256, 128, 128
HW = H * W
CO = 21
CHUNKS_PER_B = NW // B

X0 = HW
P = 64
PG = P // L
NBUF = 2
PIX_PER_W = X0 // CHUNKS_PER_B
NCHUNK = PIX_PER_W // P
CUNROLL = 2
SFLAT = C * CO
NREP = 8
PB = 2048


def _sc_partials(f, fo, oo, lab, noc_vec):
    mesh = plsc.VectorSubcoreMesh(core_axis_name="c", subcore_axis_name="s")

    @functools.partial(
        pl.kernel,
        out_type=(
            jax.ShapeDtypeStruct((NW, SFLAT), jnp.float32),
            jax.ShapeDtypeStruct((NW, 32), jnp.float32),
            jax.ShapeDtypeStruct((NW, 32), jnp.float32),
        ),
        mesh=mesh,
        compiler_params=pltpu.CompilerParams(
            needs_layout_passes=False, use_tc_tiling_on_sc=False),
        scratch_types=[
            pltpu.VMEM((NBUF, C, P), jnp.float32),
            pltpu.VMEM((NBUF, C, P), jnp.float32),
            pltpu.VMEM((NBUF, CO, P), jnp.float32),
            pltpu.VMEM((NBUF, P), jnp.int32),
            pltpu.VMEM((L,), jnp.int32),
            pltpu.VMEM((NREP * SFLAT,), jnp.float32),
            pltpu.VMEM((NREP * 32,), jnp.float32),
            pltpu.VMEM((NREP * 32,), jnp.float32),
            pltpu.VMEM((SFLAT,), jnp.float32),
            pltpu.VMEM((32,), jnp.float32),
            pltpu.VMEM((32,), jnp.float32),
            pltpu.SemaphoreType.DMA((NBUF,)),
        ],
    )
    def body(f_hbm, fo_hbm, oo_hbm, lab_hbm, noc_hbm,
             s_out, q_out, n_out,
             f_buf, fo_buf, o_buf, lab_buf, noc_buf, s_acc, q_acc, n_acc,
             s_red, q_red, n_red, sems):
        cid = lax.axis_index("c")
        sid = lax.axis_index("s")
        wid = sid * NC + cid
        bidx = wid // CHUNKS_PER_B
        hw0 = (wid % CHUNKS_PER_B) * PIX_PER_W

        zero = jnp.zeros((L,), jnp.float32)

        def zloop(i, carry):
            s_acc[pl.ds(i * L, L)] = zero
            return carry
        lax.fori_loop(0, NREP * SFLAT // L, zloop, 0)

        def zloop2(i, carry):
            q_acc[pl.ds(i * L, L)] = zero
            n_acc[pl.ds(i * L, L)] = zero
            return carry
        lax.fori_loop(0, NREP * 32 // L, zloop2, 0)

        pltpu.sync_copy(noc_hbm, noc_buf)
        noc = noc_buf[...]
        ones = jnp.full((L,), 1.0, jnp.float32)
        lane_rep = lax.iota(jnp.int32, L) % NREP

        def start_copies(slot, ci):
            off = hw0 + ci * P
            pltpu.async_copy(f_hbm.at[bidx, :, pl.ds(off, P)], f_buf.at[slot], sems.at[slot])
            pltpu.async_copy(fo_hbm.at[bidx, :, pl.ds(off, P)], fo_buf.at[slot], sems.at[slot])
            pltpu.async_copy(oo_hbm.at[bidx, :, pl.ds(off, P)], o_buf.at[slot], sems.at[slot])
            pltpu.async_copy(lab_hbm.at[bidx, pl.ds(off, P)], lab_buf.at[slot], sems.at[slot])

        def wait_copies(slot, ci):
            off = hw0 + ci * P
            pltpu.make_async_copy(f_hbm.at[bidx, :, pl.ds(off, P)], f_buf.at[slot], sems.at[slot]).wait()
            pltpu.make_async_copy(fo_hbm.at[bidx, :, pl.ds(off, P)], fo_buf.at[slot], sems.at[slot]).wait()
            pltpu.make_async_copy(oo_hbm.at[bidx, :, pl.ds(off, P)], o_buf.at[slot], sems.at[slot]).wait()
            pltpu.make_async_copy(lab_hbm.at[bidx, pl.ds(off, P)], lab_buf.at[slot], sems.at[slot]).wait()

        for s in range(NBUF):
            start_copies(s, s)

        def compute(slot, ci):
            fb, fob, ob, lb = f_buf.at[slot], fo_buf.at[slot], o_buf.at[slot], lab_buf.at[slot]
            sls = [pl.ds(pg * L, L) for pg in range(PG)]
            ms = [ob[0, sls[pg]] for pg in range(PG)]
            ks = [jnp.zeros((L,), jnp.int32) for _ in range(PG)]
            for ch in range(1, CO):
                chv = jnp.full((L,), ch, jnp.int32)
                for pg in range(PG):
                    v = ob[ch, sls[pg]]
                    upd = v > ms[pg]
                    ms[pg] = jnp.where(upd, v, ms[pg])
                    ks[pg] = jnp.where(upd, chv, ks[pg])
            for pg in range(PG):
                ks[pg] = jnp.where(lb[sls[pg]] < noc, ks[pg], 0)
                plsc.addupdate_scatter(n_acc, [ks[pg] * NREP + lane_rep], ones)
                ks[pg] = ks[pg] * NREP + lane_rep

            @plsc.parallel_loop(0, C, step=CUNROLL, unroll=4)
            def qs(cc):
                for u in range(CUNROLL):
                    c = cc + u
                    base = c * (CO * NREP)
                    for pg in range(PG):
                        sl = pl.ds(pg * L, L)
                        d = fb[c, sl] - fob[c, sl]
                        plsc.addupdate_scatter(s_acc, [ks[pg] + base], d)
                        plsc.addupdate_scatter(q_acc, [ks[pg]], d * d)

        def outer(g, carry):
            base = g * NBUF
            for s in range(NBUF):
                ci = base + s
                wait_copies(s, ci)
                compute(s, ci)

                @pl.when(ci + NBUF < NCHUNK)
                def _():
                    start_copies(s, ci + NBUF)
            return carry
        lax.fori_loop(0, NCHUNK // NBUF, outer, 0)

        base16 = lax.iota(jnp.int32, L) * NREP

        def red(i, carry):
            idx0 = i * (L * NREP) + base16
            acc = plsc.load_gather(s_acc, [idx0])
            for r in range(1, NREP):
                acc = acc + plsc.load_gather(s_acc, [idx0 + r])
            s_red[pl.ds(i * L, L)] = acc
            return carry
        lax.fori_loop(0, SFLAT // L, red, 0)

        def redqn(i, carry):
            idx0 = i * (L * NREP) + base16
            qa = plsc.load_gather(q_acc, [idx0])
            na = plsc.load_gather(n_acc, [idx0])
            for r in range(1, NREP):
                qa = qa + plsc.load_gather(q_acc, [idx0 + r])
                na = na + plsc.load_gather(n_acc, [idx0 + r])
            q_red[pl.ds(i * L, L)] = qa
            n_red[pl.ds(i * L, L)] = na
            return carry
        lax.fori_loop(0, 32 // L, redqn, 0)

        pltpu.sync_copy(s_red, s_out.at[wid])
        pltpu.sync_copy(q_red, q_out.at[wid])
        pltpu.sync_copy(n_red, n_out.at[wid])

    return body(f, fo, oo, lab, noc_vec)


def _tc_partials_body(noc_ref, f_ref, fo_ref, oo_ref, lab_ref,
                      s_ref, q_ref, n_ref, s_scr, q_scr, n_scr):
    step = pl.program_id(0)

    @pl.when(step == 0)
    def _():
        s_scr[...] = jnp.zeros_like(s_scr)
        q_scr[...] = jnp.zeros_like(q_scr)
        n_scr[...] = jnp.zeros_like(n_scr)

    d = f_ref[0] - fo_ref[0]
    oo = oo_ref[0]
    m = jnp.max(oo, axis=0, keepdims=True)
    chi = lax.broadcasted_iota(jnp.int32, (CO, PB), 0)
    idx = jnp.min(jnp.where(oo == m, chi, CO), axis=0, keepdims=True)
    lab = lab_ref[0]
    idx = jnp.where(lab < noc_ref[0], idx, 0)
    onehot_t = (chi == idx).astype(jnp.float32)

    s_scr[...] += lax.dot_general(d, onehot_t, (((1,), (1,)), ((), ())),
                                  preferred_element_type=jnp.float32)
    rowsq = jnp.sum(d * d, axis=0, keepdims=True)
    q_scr[...] += lax.dot_general(rowsq, onehot_t, (((1,), (1,)), ((), ())),
                                  preferred_element_type=jnp.float32)
    ones_row = jnp.ones((1, PB), jnp.float32)
    n_scr[...] += lax.dot_general(ones_row, onehot_t, (((1,), (1,)), ((), ())),
                                  preferred_element_type=jnp.float32)

    @pl.when(step == pl.num_programs(0) - 1)
    def _():
        s_ref[...] = s_scr[...]
        q_ref[...] = q_scr[...]
        n_ref[...] = n_scr[...]


def _tc_partials(f, fo, oo, lab3, noc11):
    nblk = (HW - X0) // PB
    steps = B * nblk

    def bmap(i):
        return i // nblk

    def pmap(i):
        return X0 // PB + i % nblk

    return pl.pallas_call(
        _tc_partials_body,
        grid=(steps,),
        in_specs=[
            pl.BlockSpec(memory_space=pltpu.SMEM),
            pl.BlockSpec((1, C, PB), lambda i: (bmap(i), 0, pmap(i))),
            pl.BlockSpec((1, C, PB), lambda i: (bmap(i), 0, pmap(i))),
            pl.BlockSpec((1, CO, PB), lambda i: (bmap(i), 0, pmap(i))),
            pl.BlockSpec((1, 1, PB), lambda i: (bmap(i), 0, pmap(i))),
        ],
        out_specs=[
            pl.BlockSpec((C, CO), lambda i: (0, 0)),
            pl.BlockSpec((1, CO), lambda i: (0, 0)),
            pl.BlockSpec((1, CO), lambda i: (0, 0)),
        ],
        out_shape=[
            jax.ShapeDtypeStruct((C, CO), jnp.float32),
            jax.ShapeDtypeStruct((1, CO), jnp.float32),
            jax.ShapeDtypeStruct((1, CO), jnp.float32),
        ],
        scratch_shapes=[
            pltpu.VMEM((C, CO), jnp.float32),
            pltpu.VMEM((1, CO), jnp.float32),
            pltpu.VMEM((1, CO), jnp.float32),
        ],
    )(noc11, f, fo, oo, lab3)


HAS_SC = X0 > 0
HAS_TC = X0 < HW


def _combine_body(*refs):
    i = 0
    st = jnp.zeros((C, CO), jnp.float32)
    q = jnp.zeros((1, CO), jnp.float32)
    n = jnp.zeros((1, CO), jnp.float32)
    if HAS_SC:
        s_sc, q_sc, n_sc = refs[0], refs[1], refs[2]
        i = 3
        st = st + jnp.sum(s_sc[...], axis=0)
        q = q + jnp.sum(q_sc[...], axis=0, keepdims=True)[:, :CO]
        n = n + jnp.sum(n_sc[...], axis=0, keepdims=True)[:, :CO]
    if HAS_TC:
        st = st + refs[i][...]
        q = q + refs[i + 1][...]
        n = n + refs[i + 2][...]
    o_ref = refs[-1]
    ss = jnp.sum(st * st, axis=0, keepdims=True)
    cls = lax.broadcasted_iota(jnp.int32, (1, CO), 1)
    denom = jnp.maximum(n, 1.0)
    loss_cl = q / denom - ss / (denom * denom)
    valid = (cls >= 1) & (n > 0.0)
    total = jnp.sum(jnp.where(valid, loss_cl, 0.0))
    present = jnp.sum(jnp.where(valid, 1.0, 0.0))
    loss = jnp.where(present > 0.0, total / jnp.maximum(present, 1.0), 0.0)
    o_ref[...] = jnp.reshape(loss, (1, 1))


def kernel(features, features_old, outputs_old, labels, prototypes, num_old_class):
    del prototypes
    f = features.reshape(B, C, HW)
    fo = features_old.reshape(B, C, HW)
    oo = outputs_old.reshape(B, CO, HW)
    lab = labels.reshape(B, HW)

    operands = []
    if HAS_SC:
        noc_vec = jnp.full((L,), num_old_class, jnp.int32)
        s_sc, q_sc, n_sc = _sc_partials(f, fo, oo, lab, noc_vec)
        operands += [s_sc.reshape(NW, C, CO), q_sc, n_sc]
    if HAS_TC:
        noc11 = jnp.asarray(num_old_class, jnp.int32).reshape(1)
        s_tc, q_tc, n_tc = _tc_partials(f, fo, oo, lab.reshape(B, 1, HW), noc11)
        operands += [s_tc, q_tc, n_tc]

    out = pl.pallas_call(
        _combine_body,
        out_shape=jax.ShapeDtypeStruct((1, 1), jnp.float32),
    )(*operands)
    return out[0, 0]


# speedup vs baseline: 1.1751x; 1.1751x over previous
import functools

import jax
import jax.numpy as jnp
from jax import lax
from jax.experimental import pallas as pl
from jax.experimental.pallas import tpu as pltpu
from jax.experimental.pallas import tpu_sc as plsc

NC, NS, L = 2, 16, 16
NW = NC * NS
B, C, H, W = 8, 256, 128, 128
HW = H * W
CO = 21
CHUNKS_PER_B = NW // B
X0 = HW
PIX_PER_W = X0 // CHUNKS_PER_B
PA = 128
NACH = PIX_PER_W // PA
CB = 4
NBBLK = C // CB
NBUF = 2
NREP = 8
SFLAT = C * CO
PB = 2048


def _sc_partials(f, fo, oo, lab, noc_vec):
    mesh = plsc.VectorSubcoreMesh(core_axis_name="c", subcore_axis_name="s")

    @functools.partial(
        pl.kernel,
        out_type=(
            jax.ShapeDtypeStruct((NW, SFLAT), jnp.float32),
            jax.ShapeDtypeStruct((NW, 32), jnp.float32),
            jax.ShapeDtypeStruct((NW, 32), jnp.float32),
        ),
        mesh=mesh,
        compiler_params=pltpu.CompilerParams(
            needs_layout_passes=False, use_tc_tiling_on_sc=False),
        scratch_types=[
            pltpu.VMEM((NBUF, CB, PIX_PER_W), jnp.float32),
            pltpu.VMEM((NBUF, CB, PIX_PER_W), jnp.float32),
            pltpu.VMEM((NBUF, CO, PA), jnp.float32),
            pltpu.VMEM((NBUF, PA), jnp.int32),
            pltpu.VMEM((PIX_PER_W,), jnp.int32),
            pltpu.VMEM((L,), jnp.int32),
            pltpu.VMEM((NREP * SFLAT,), jnp.float32),
            pltpu.VMEM((NREP * 32,), jnp.float32),
            pltpu.VMEM((NREP * 32,), jnp.float32),
            pltpu.VMEM((SFLAT,), jnp.float32),
            pltpu.VMEM((32,), jnp.float32),
            pltpu.VMEM((32,), jnp.float32),
            pltpu.SemaphoreType.DMA((NBUF,)),
            pltpu.SemaphoreType.DMA((NBUF,)),
        ],
    )
    def body(f_hbm, fo_hbm, oo_hbm, lab_hbm, noc_hbm,
             s_out, q_out, n_out,
             f_buf, fo_buf, o_buf, lab_buf, cls_buf, noc_buf,
             s_acc, q_acc, n_acc, s_red, q_red, n_red, sems_a, sems_b):
        cid = lax.axis_index("c")
        sid = lax.axis_index("s")
        wid = sid * NC + cid
        bidx = wid // CHUNKS_PER_B
        hw0 = (wid % CHUNKS_PER_B) * PIX_PER_W

        zero = jnp.zeros((L,), jnp.float32)

        def zloop(i, carry):
            s_acc[pl.ds(i * L, L)] = zero
            return carry
        lax.fori_loop(0, NREP * SFLAT // L, zloop, 0)

        def zloop2(i, carry):
            q_acc[pl.ds(i * L, L)] = zero
            n_acc[pl.ds(i * L, L)] = zero
            return carry
        lax.fori_loop(0, NREP * 32 // L, zloop2, 0)

        pltpu.sync_copy(noc_hbm, noc_buf)
        noc = noc_buf[...]
        ones = jnp.full((L,), 1.0, jnp.float32)
        lane_rep = lax.iota(jnp.int32, L) % NREP

        def start_a(slot, ci):
            off = hw0 + ci * PA
            pltpu.async_copy(oo_hbm.at[bidx, :, pl.ds(off, PA)], o_buf.at[slot], sems_a.at[slot])
            pltpu.async_copy(lab_hbm.at[bidx, pl.ds(off, PA)], lab_buf.at[slot], sems_a.at[slot])

        def wait_a(slot, ci):
            off = hw0 + ci * PA
            pltpu.make_async_copy(oo_hbm.at[bidx, :, pl.ds(off, PA)], o_buf.at[slot], sems_a.at[slot]).wait()
            pltpu.make_async_copy(lab_hbm.at[bidx, pl.ds(off, PA)], lab_buf.at[slot], sems_a.at[slot]).wait()

        for s in range(NBUF):
            start_a(s, s)

        def achunk(slot, ci):
            ob, lb = o_buf.at[slot], lab_buf.at[slot]

            @plsc.parallel_loop(0, PA // L, step=1, unroll=2)
            def _(pg):
                sl = pl.ds(pg * L, L)
                m = ob[0, sl]
                k = jnp.zeros((L,), jnp.int32)
                for ch in range(1, CO):
                    v = ob[ch, sl]
                    upd = v > m
                    m = jnp.where(upd, v, m)
                    k = jnp.where(upd, jnp.full((L,), ch, jnp.int32), k)
                k = jnp.where(lb[sl] < noc, k, 0)
                k = k * NREP + lane_rep
                plsc.addupdate_scatter(n_acc, [k], ones)
                cls_buf[pl.ds(ci * PA + pg * L, L)] = k

        def outer_a(g, carry):
            base = g * NBUF
            for s in range(NBUF):
                ci = base + s
                wait_a(s, ci)
                achunk(s, ci)

                @pl.when(ci + NBUF < NACH)
                def _():
                    start_a(s, ci + NBUF)
            return carry
        lax.fori_loop(0, NACH // NBUF, outer_a, 0)

        def start_b(slot, cb):
            c0 = cb * CB
            pltpu.async_copy(f_hbm.at[bidx, pl.ds(c0, CB), pl.ds(hw0, PIX_PER_W)], f_buf.at[slot], sems_b.at[slot])
            pltpu.async_copy(fo_hbm.at[bidx, pl.ds(c0, CB), pl.ds(hw0, PIX_PER_W)], fo_buf.at[slot], sems_b.at[slot])

        def wait_b(slot, cb):
            c0 = cb * CB
            pltpu.make_async_copy(f_hbm.at[bidx, pl.ds(c0, CB), pl.ds(hw0, PIX_PER_W)], f_buf.at[slot], sems_b.at[slot]).wait()
            pltpu.make_async_copy(fo_hbm.at[bidx, pl.ds(c0, CB), pl.ds(hw0, PIX_PER_W)], fo_buf.at[slot], sems_b.at[slot]).wait()

        for s in range(NBUF):
            start_b(s, s)

        def bblock(slot, cb):
            fb, fob = f_buf.at[slot], fo_buf.at[slot]
            cbase = cb * CB * (CO * NREP)

            @plsc.parallel_loop(0, PIX_PER_W // L, step=1, unroll=2)
            def _(pg):
                sl = pl.ds(pg * L, L)
                k = cls_buf[sl]
                qp = None
                for u in range(CB):
                    d = fb[u, sl] - fob[u, sl]
                    plsc.addupdate_scatter(s_acc, [k + (cbase + u * (CO * NREP))], d)
                    qp = d * d if qp is None else qp + d * d
                plsc.addupdate_scatter(q_acc, [k], qp)

        def outer_b(g, carry):
            base = g * NBUF
            for s in range(NBUF):
                cb = base + s
                wait_b(s, cb)
                bblock(s, cb)

                @pl.when(cb + NBUF < NBBLK)
                def _():
                    start_b(s, cb + NBUF)
            return carry
        lax.fori_loop(0, NBBLK // NBUF, outer_b, 0)

        base16 = lax.iota(jnp.int32, L) * NREP

        def red(i, carry):
            idx0 = i * (L * NREP) + base16
            acc = plsc.load_gather(s_acc, [idx0])
            for r in range(1, NREP):
                acc = acc + plsc.load_gather(s_acc, [idx0 + r])
            s_red[pl.ds(i * L, L)] = acc
            return carry
        lax.fori_loop(0, SFLAT // L, red, 0)

        def redqn(i, carry):
            idx0 = i * (L * NREP) + base16
            qa = plsc.load_gather(q_acc, [idx0])
            na = plsc.load_gather(n_acc, [idx0])
            for r in range(1, NREP):
                qa = qa + plsc.load_gather(q_acc, [idx0 + r])
                na = na + plsc.load_gather(n_acc, [idx0 + r])
            q_red[pl.ds(i * L, L)] = qa
            n_red[pl.ds(i * L, L)] = na
            return carry
        lax.fori_loop(0, 32 // L, redqn, 0)

        pltpu.sync_copy(s_red, s_out.at[wid])
        pltpu.sync_copy(q_red, q_out.at[wid])
        pltpu.sync_copy(n_red, n_out.at[wid])

    return body(f, fo, oo, lab, noc_vec)


def _tc_partials_body(noc_ref, f_ref, fo_ref, oo_ref, lab_ref,
                      s_ref, q_ref, n_ref, s_scr, q_scr, n_scr):
    step = pl.program_id(0)

    @pl.when(step == 0)
    def _():
        s_scr[...] = jnp.zeros_like(s_scr)
        q_scr[...] = jnp.zeros_like(q_scr)
        n_scr[...] = jnp.zeros_like(n_scr)

    d = f_ref[0] - fo_ref[0]
    oo = oo_ref[0]
    m = jnp.max(oo, axis=0, keepdims=True)
    chi = lax.broadcasted_iota(jnp.int32, (CO, PB), 0)
    idx = jnp.min(jnp.where(oo == m, chi, CO), axis=0, keepdims=True)
    lab = lab_ref[0]
    idx = jnp.where(lab < noc_ref[0], idx, 0)
    onehot_t = (chi == idx).astype(jnp.float32)

    s_scr[...] += lax.dot_general(d, onehot_t, (((1,), (1,)), ((), ())),
                                  preferred_element_type=jnp.float32)
    rowsq = jnp.sum(d * d, axis=0, keepdims=True)
    q_scr[...] += lax.dot_general(rowsq, onehot_t, (((1,), (1,)), ((), ())),
                                  preferred_element_type=jnp.float32)
    ones_row = jnp.ones((1, PB), jnp.float32)
    n_scr[...] += lax.dot_general(ones_row, onehot_t, (((1,), (1,)), ((), ())),
                                  preferred_element_type=jnp.float32)

    @pl.when(step == pl.num_programs(0) - 1)
    def _():
        s_ref[...] = s_scr[...]
        q_ref[...] = q_scr[...]
        n_ref[...] = n_scr[...]


def _tc_partials(f, fo, oo, lab3, noc11):
    nblk = (HW - X0) // PB
    steps = B * nblk

    def bmap(i):
        return i // nblk

    def pmap(i):
        return X0 // PB + i % nblk

    return pl.pallas_call(
        _tc_partials_body,
        grid=(steps,),
        in_specs=[
            pl.BlockSpec(memory_space=pltpu.SMEM),
            pl.BlockSpec((1, C, PB), lambda i: (bmap(i), 0, pmap(i))),
            pl.BlockSpec((1, C, PB), lambda i: (bmap(i), 0, pmap(i))),
            pl.BlockSpec((1, CO, PB), lambda i: (bmap(i), 0, pmap(i))),
            pl.BlockSpec((1, 1, PB), lambda i: (bmap(i), 0, pmap(i))),
        ],
        out_specs=[
            pl.BlockSpec((C, CO), lambda i: (0, 0)),
            pl.BlockSpec((1, CO), lambda i: (0, 0)),
            pl.BlockSpec((1, CO), lambda i: (0, 0)),
        ],
        out_shape=[
            jax.ShapeDtypeStruct((C, CO), jnp.float32),
            jax.ShapeDtypeStruct((1, CO), jnp.float32),
            jax.ShapeDtypeStruct((1, CO), jnp.float32),
        ],
        scratch_shapes=[
            pltpu.VMEM((C, CO), jnp.float32),
            pltpu.VMEM((1, CO), jnp.float32),
            pltpu.VMEM((1, CO), jnp.float32),
        ],
    )(noc11, f, fo, oo, lab3)


HAS_SC = X0 > 0
HAS_TC = X0 < HW


def _combine_body(*refs):
    i = 0
    st = jnp.zeros((C, CO), jnp.float32)
    q = jnp.zeros((1, CO), jnp.float32)
    n = jnp.zeros((1, CO), jnp.float32)
    if HAS_SC:
        s_sc, q_sc, n_sc = refs[0], refs[1], refs[2]
        i = 3
        st = st + jnp.sum(s_sc[...], axis=0)
        q = q + jnp.sum(q_sc[...], axis=0, keepdims=True)[:, :CO]
        n = n + jnp.sum(n_sc[...], axis=0, keepdims=True)[:, :CO]
    if HAS_TC:
        st = st + refs[i][...]
        q = q + refs[i + 1][...]
        n = n + refs[i + 2][...]
    o_ref = refs[-1]
    ss = jnp.sum(st * st, axis=0, keepdims=True)
    cls = lax.broadcasted_iota(jnp.int32, (1, CO), 1)
    denom = jnp.maximum(n, 1.0)
    loss_cl = q / denom - ss / (denom * denom)
    valid = (cls >= 1) & (n > 0.0)
    total = jnp.sum(jnp.where(valid, loss_cl, 0.0))
    present = jnp.sum(jnp.where(valid, 1.0, 0.0))
    loss = jnp.where(present > 0.0, total / jnp.maximum(present, 1.0), 0.0)
    o_ref[...] = jnp.reshape(loss, (1, 1))


def kernel(features, features_old, outputs_old, labels, prototypes, num_old_class):
    del prototypes
    f = features.reshape(B, C, HW)
    fo = features_old.reshape(B, C, HW)
    oo = outputs_old.reshape(B, CO, HW)
    lab = labels.reshape(B, HW)

    operands = []
    if HAS_SC:
        noc_vec = jnp.full((L,), num_old_class, jnp.int32)
        s_sc, q_sc, n_sc = _sc_partials(f, fo, oo, lab, noc_vec)
        operands += [s_sc.reshape(NW, C, CO), q_sc, n_sc]
    if HAS_TC:
        noc11 = jnp.asarray(num_old_class, jnp.int32).reshape(1)
        s_tc, q_tc, n_tc = _tc_partials(f, fo, oo, lab.reshape(B, 1, HW), noc11)
        operands += [s_tc, q_tc, n_tc]

    out = pl.pallas_call(
        _combine_body,
        out_shape=jax.ShapeDtypeStruct((1, 1), jnp.float32),
    )(*operands)
    return out[0, 0]


# speedup vs baseline: 1.3270x; 1.1292x over previous
import functools

import jax
import jax.numpy as jnp
from jax import lax
from jax.experimental import pallas as pl
from jax.experimental.pallas import tpu as pltpu
from jax.experimental.pallas import tpu_sc as plsc

NC, NS, L = 2, 16, 16
NW = NC * NS
B, C, H, W = 8, 256, 128, 128
HW = H * W
CO = 21
CHUNKS_PER_B = NW // B

X0 = HW
P = 64
PG = P // L
NBUF = 2
PIX_PER_W = X0 // CHUNKS_PER_B
NCHUNK = PIX_PER_W // P
CUNROLL = 2
SFLAT = C * CO
NREP = 8
PB = 2048


def _sc_partials(f, fo, oo, lab, noc_vec):
    mesh = plsc.VectorSubcoreMesh(core_axis_name="c", subcore_axis_name="s")

    @functools.partial(
        pl.kernel,
        out_type=(
            jax.ShapeDtypeStruct((NW, SFLAT), jnp.float32),
            jax.ShapeDtypeStruct((NW, 32), jnp.float32),
            jax.ShapeDtypeStruct((NW, 32), jnp.float32),
        ),
        mesh=mesh,
        compiler_params=pltpu.CompilerParams(
            needs_layout_passes=False, use_tc_tiling_on_sc=False),
        scratch_types=[
            pltpu.VMEM((NBUF, C, P), jnp.float32),
            pltpu.VMEM((NBUF, C, P), jnp.float32),
            pltpu.VMEM((NBUF, CO, P), jnp.float32),
            pltpu.VMEM((NBUF, P), jnp.int32),
            pltpu.VMEM((L,), jnp.int32),
            pltpu.VMEM((NREP * SFLAT,), jnp.float32),
            pltpu.VMEM((NREP * 32,), jnp.float32),
            pltpu.VMEM((NREP * 32,), jnp.float32),
            pltpu.VMEM((SFLAT,), jnp.float32),
            pltpu.VMEM((32,), jnp.float32),
            pltpu.VMEM((32,), jnp.float32),
            pltpu.SemaphoreType.DMA((NBUF,)),
        ],
    )
    def body(f_hbm, fo_hbm, oo_hbm, lab_hbm, noc_hbm,
             s_out, q_out, n_out,
             f_buf, fo_buf, o_buf, lab_buf, noc_buf, s_acc, q_acc, n_acc,
             s_red, q_red, n_red, sems):
        cid = lax.axis_index("c")
        sid = lax.axis_index("s")
        wid = sid * NC + cid
        bidx = wid // CHUNKS_PER_B
        hw0 = (wid % CHUNKS_PER_B) * PIX_PER_W

        zero = jnp.zeros((L,), jnp.float32)

        def zloop(i, carry):
            s_acc[pl.ds(i * L, L)] = zero
            return carry
        lax.fori_loop(0, NREP * SFLAT // L, zloop, 0)

        def zloop2(i, carry):
            q_acc[pl.ds(i * L, L)] = zero
            n_acc[pl.ds(i * L, L)] = zero
            return carry
        lax.fori_loop(0, NREP * 32 // L, zloop2, 0)

        pltpu.sync_copy(noc_hbm, noc_buf)
        noc = noc_buf[...]
        ones = jnp.full((L,), 1.0, jnp.float32)
        lane_rep = lax.iota(jnp.int32, L) % NREP

        def start_copies(slot, ci):
            off = hw0 + ci * P
            pltpu.async_copy(f_hbm.at[bidx, :, pl.ds(off, P)], f_buf.at[slot], sems.at[slot])
            pltpu.async_copy(fo_hbm.at[bidx, :, pl.ds(off, P)], fo_buf.at[slot], sems.at[slot])
            pltpu.async_copy(oo_hbm.at[bidx, :, pl.ds(off, P)], o_buf.at[slot], sems.at[slot])
            pltpu.async_copy(lab_hbm.at[bidx, pl.ds(off, P)], lab_buf.at[slot], sems.at[slot])

        def wait_copies(slot, ci):
            off = hw0 + ci * P
            pltpu.make_async_copy(f_hbm.at[bidx, :, pl.ds(off, P)], f_buf.at[slot], sems.at[slot]).wait()
            pltpu.make_async_copy(fo_hbm.at[bidx, :, pl.ds(off, P)], fo_buf.at[slot], sems.at[slot]).wait()
            pltpu.make_async_copy(oo_hbm.at[bidx, :, pl.ds(off, P)], o_buf.at[slot], sems.at[slot]).wait()
            pltpu.make_async_copy(lab_hbm.at[bidx, pl.ds(off, P)], lab_buf.at[slot], sems.at[slot]).wait()

        for s in range(NBUF):
            start_copies(s, s)

        def compute(slot, ci):
            fb, fob, ob, lb = f_buf.at[slot], fo_buf.at[slot], o_buf.at[slot], lab_buf.at[slot]
            sls = [pl.ds(pg * L, L) for pg in range(PG)]
            ms = [ob[0, sls[pg]] for pg in range(PG)]
            ks = [jnp.zeros((L,), jnp.int32) for _ in range(PG)]
            for ch in range(1, CO):
                chv = jnp.full((L,), ch, jnp.int32)
                for pg in range(PG):
                    v = ob[ch, sls[pg]]
                    upd = v > ms[pg]
                    ms[pg] = jnp.where(upd, v, ms[pg])
                    ks[pg] = jnp.where(upd, chv, ks[pg])
            for pg in range(PG):
                ks[pg] = jnp.where(lb[sls[pg]] < noc, ks[pg], 0)
                plsc.addupdate_scatter(n_acc, [ks[pg] * NREP + lane_rep], ones)
                ks[pg] = ks[pg] * NREP + lane_rep

            q0 = tuple(jnp.zeros((L,), jnp.float32) for _ in range(PG))

            @plsc.parallel_loop(0, C, step=CUNROLL, unroll=8, carry=q0)
            def qs(cc, qcarry):
                out = list(qcarry)
                for u in range(CUNROLL):
                    c = cc + u
                    base = c * (CO * NREP)
                    for pg in range(PG):
                        sl = pl.ds(pg * L, L)
                        d = fb[c, sl] - fob[c, sl]
                        plsc.addupdate_scatter(s_acc, [ks[pg] + base], d)
                        out[pg] = out[pg] + d * d
                return tuple(out)
            for pg in range(PG):
                plsc.addupdate_scatter(q_acc, [ks[pg]], qs[pg])

        def outer(g, carry):
            base = g * NBUF
            for s in range(NBUF):
                ci = base + s
                wait_copies(s, ci)
                compute(s, ci)

                @pl.when(ci + NBUF < NCHUNK)
                def _():
                    start_copies(s, ci + NBUF)
            return carry
        lax.fori_loop(0, NCHUNK // NBUF, outer, 0)

        base16 = lax.iota(jnp.int32, L) * NREP

        def red(i, carry):
            idx0 = i * (L * NREP) + base16
            acc = plsc.load_gather(s_acc, [idx0])
            for r in range(1, NREP):
                acc = acc + plsc.load_gather(s_acc, [idx0 + r])
            s_red[pl.ds(i * L, L)] = acc
            return carry
        lax.fori_loop(0, SFLAT // L, red, 0)

        def redqn(i, carry):
            idx0 = i * (L * NREP) + base16
            qa = plsc.load_gather(q_acc, [idx0])
            na = plsc.load_gather(n_acc, [idx0])
            for r in range(1, NREP):
                qa = qa + plsc.load_gather(q_acc, [idx0 + r])
                na = na + plsc.load_gather(n_acc, [idx0 + r])
            q_red[pl.ds(i * L, L)] = qa
            n_red[pl.ds(i * L, L)] = na
            return carry
        lax.fori_loop(0, 32 // L, redqn, 0)

        pltpu.sync_copy(s_red, s_out.at[wid])
        pltpu.sync_copy(q_red, q_out.at[wid])
        pltpu.sync_copy(n_red, n_out.at[wid])

    return body(f, fo, oo, lab, noc_vec)


def _tc_partials_body(noc_ref, f_ref, fo_ref, oo_ref, lab_ref,
                      s_ref, q_ref, n_ref, s_scr, q_scr, n_scr):
    step = pl.program_id(0)

    @pl.when(step == 0)
    def _():
        s_scr[...] = jnp.zeros_like(s_scr)
        q_scr[...] = jnp.zeros_like(q_scr)
        n_scr[...] = jnp.zeros_like(n_scr)

    d = f_ref[0] - fo_ref[0]
    oo = oo_ref[0]
    m = jnp.max(oo, axis=0, keepdims=True)
    chi = lax.broadcasted_iota(jnp.int32, (CO, PB), 0)
    idx = jnp.min(jnp.where(oo == m, chi, CO), axis=0, keepdims=True)
    lab = lab_ref[0]
    idx = jnp.where(lab < noc_ref[0], idx, 0)
    onehot_t = (chi == idx).astype(jnp.float32)

    s_scr[...] += lax.dot_general(d, onehot_t, (((1,), (1,)), ((), ())),
                                  preferred_element_type=jnp.float32)
    rowsq = jnp.sum(d * d, axis=0, keepdims=True)
    q_scr[...] += lax.dot_general(rowsq, onehot_t, (((1,), (1,)), ((), ())),
                                  preferred_element_type=jnp.float32)
    ones_row = jnp.ones((1, PB), jnp.float32)
    n_scr[...] += lax.dot_general(ones_row, onehot_t, (((1,), (1,)), ((), ())),
                                  preferred_element_type=jnp.float32)

    @pl.when(step == pl.num_programs(0) - 1)
    def _():
        s_ref[...] = s_scr[...]
        q_ref[...] = q_scr[...]
        n_ref[...] = n_scr[...]


def _tc_partials(f, fo, oo, lab3, noc11):
    nblk = (HW - X0) // PB
    steps = B * nblk

    def bmap(i):
        return i // nblk

    def pmap(i):
        return X0 // PB + i % nblk

    return pl.pallas_call(
        _tc_partials_body,
        grid=(steps,),
        in_specs=[
            pl.BlockSpec(memory_space=pltpu.SMEM),
            pl.BlockSpec((1, C, PB), lambda i: (bmap(i), 0, pmap(i))),
            pl.BlockSpec((1, C, PB), lambda i: (bmap(i), 0, pmap(i))),
            pl.BlockSpec((1, CO, PB), lambda i: (bmap(i), 0, pmap(i))),
            pl.BlockSpec((1, 1, PB), lambda i: (bmap(i), 0, pmap(i))),
        ],
        out_specs=[
            pl.BlockSpec((C, CO), lambda i: (0, 0)),
            pl.BlockSpec((1, CO), lambda i: (0, 0)),
            pl.BlockSpec((1, CO), lambda i: (0, 0)),
        ],
        out_shape=[
            jax.ShapeDtypeStruct((C, CO), jnp.float32),
            jax.ShapeDtypeStruct((1, CO), jnp.float32),
            jax.ShapeDtypeStruct((1, CO), jnp.float32),
        ],
        scratch_shapes=[
            pltpu.VMEM((C, CO), jnp.float32),
            pltpu.VMEM((1, CO), jnp.float32),
            pltpu.VMEM((1, CO), jnp.float32),
        ],
    )(noc11, f, fo, oo, lab3)


HAS_SC = X0 > 0
HAS_TC = X0 < HW


def _combine_body(*refs):
    i = 0
    st = jnp.zeros((C, CO), jnp.float32)
    q = jnp.zeros((1, CO), jnp.float32)
    n = jnp.zeros((1, CO), jnp.float32)
    if HAS_SC:
        s_sc, q_sc, n_sc = refs[0], refs[1], refs[2]
        i = 3
        st = st + jnp.sum(s_sc[...], axis=0)
        q = q + jnp.sum(q_sc[...], axis=0, keepdims=True)[:, :CO]
        n = n + jnp.sum(n_sc[...], axis=0, keepdims=True)[:, :CO]
    if HAS_TC:
        st = st + refs[i][...]
        q = q + refs[i + 1][...]
        n = n + refs[i + 2][...]
    o_ref = refs[-1]
    ss = jnp.sum(st * st, axis=0, keepdims=True)
    cls = lax.broadcasted_iota(jnp.int32, (1, CO), 1)
    denom = jnp.maximum(n, 1.0)
    loss_cl = q / denom - ss / (denom * denom)
    valid = (cls >= 1) & (n > 0.0)
    total = jnp.sum(jnp.where(valid, loss_cl, 0.0))
    present = jnp.sum(jnp.where(valid, 1.0, 0.0))
    loss = jnp.where(present > 0.0, total / jnp.maximum(present, 1.0), 0.0)
    o_ref[...] = jnp.reshape(loss, (1, 1))


def kernel(features, features_old, outputs_old, labels, prototypes, num_old_class):
    del prototypes
    f = features.reshape(B, C, HW)
    fo = features_old.reshape(B, C, HW)
    oo = outputs_old.reshape(B, CO, HW)
    lab = labels.reshape(B, HW)

    operands = []
    if HAS_SC:
        noc_vec = jnp.full((L,), num_old_class, jnp.int32)
        s_sc, q_sc, n_sc = _sc_partials(f, fo, oo, lab, noc_vec)
        operands += [s_sc.reshape(NW, C, CO), q_sc, n_sc]
    if HAS_TC:
        noc11 = jnp.asarray(num_old_class, jnp.int32).reshape(1)
        s_tc, q_tc, n_tc = _tc_partials(f, fo, oo, lab.reshape(B, 1, HW), noc11)
        operands += [s_tc, q_tc, n_tc]

    out = pl.pallas_call(
        _combine_body,
        out_shape=jax.ShapeDtypeStruct((1, 1), jnp.float32),
    )(*operands)
    return out[0, 0]


# speedup vs baseline: 1.4496x; 1.0925x over previous
import functools

import jax
import jax.numpy as jnp
from jax import lax
from jax.experimental import pallas as pl
from jax.experimental.pallas import tpu as pltpu
from jax.experimental.pallas import tpu_sc as plsc

NC, NS, L = 2, 16, 16
NW = NC * NS
B, C, H, W = 8, 256, 128, 128
HW = H * W
CO = 21
CHUNKS_PER_B = NW // B

X0 = HW
P = 64
PG = P // L
NBUF = 2
PIX_PER_W = X0 // CHUNKS_PER_B
NCHUNK = PIX_PER_W // P
CUNROLL = 2
SFLAT = C * CO
NREP = 8
PB = 2048


def _sc_partials(f, fo, oo, lab, noc_vec):
    mesh = plsc.VectorSubcoreMesh(core_axis_name="c", subcore_axis_name="s")

    @functools.partial(
        pl.kernel,
        out_type=(
            jax.ShapeDtypeStruct((NW, SFLAT), jnp.float32),
            jax.ShapeDtypeStruct((NW, 32), jnp.float32),
            jax.ShapeDtypeStruct((NW, 32), jnp.float32),
        ),
        mesh=mesh,
        compiler_params=pltpu.CompilerParams(
            needs_layout_passes=False, use_tc_tiling_on_sc=False),
        scratch_types=[
            pltpu.VMEM((NBUF, C, P), jnp.float32),
            pltpu.VMEM((NBUF, C, P), jnp.float32),
            pltpu.VMEM((NBUF, CO, P), jnp.float32),
            pltpu.VMEM((NBUF, P), jnp.int32),
            pltpu.VMEM((L,), jnp.int32),
            pltpu.VMEM((NREP * SFLAT,), jnp.float32),
            pltpu.VMEM((NREP * 32,), jnp.float32),
            pltpu.VMEM((NREP * 32,), jnp.float32),
            pltpu.VMEM((SFLAT,), jnp.float32),
            pltpu.VMEM((32,), jnp.float32),
            pltpu.VMEM((32,), jnp.float32),
            pltpu.SemaphoreType.DMA((NBUF,)),
        ],
    )
    def body(f_hbm, fo_hbm, oo_hbm, lab_hbm, noc_hbm,
             s_out, q_out, n_out,
             f_buf, fo_buf, o_buf, lab_buf, noc_buf, s_acc, q_acc, n_acc,
             s_red, q_red, n_red, sems):
        cid = lax.axis_index("c")
        sid = lax.axis_index("s")
        wid = sid * NC + cid
        bidx = wid // CHUNKS_PER_B
        hw0 = (wid % CHUNKS_PER_B) * PIX_PER_W

        zero = jnp.zeros((L,), jnp.float32)

        @plsc.parallel_loop(0, NREP * SFLAT // L, step=1, unroll=8)
        def _(i):
            s_acc[pl.ds(i * L, L)] = zero

        @plsc.parallel_loop(0, NREP * 32 // L, step=1, unroll=2)
        def _(i):
            q_acc[pl.ds(i * L, L)] = zero
            n_acc[pl.ds(i * L, L)] = zero

        pltpu.sync_copy(noc_hbm, noc_buf)
        noc = noc_buf[...]
        ones = jnp.full((L,), 1.0, jnp.float32)
        lane_rep = lax.iota(jnp.int32, L) % NREP

        def start_copies(slot, ci):
            off = hw0 + ci * P
            pltpu.async_copy(f_hbm.at[bidx, :, pl.ds(off, P)], f_buf.at[slot], sems.at[slot])
            pltpu.async_copy(fo_hbm.at[bidx, :, pl.ds(off, P)], fo_buf.at[slot], sems.at[slot])
            pltpu.async_copy(oo_hbm.at[bidx, :, pl.ds(off, P)], o_buf.at[slot], sems.at[slot])
            pltpu.async_copy(lab_hbm.at[bidx, pl.ds(off, P)], lab_buf.at[slot], sems.at[slot])

        def wait_copies(slot, ci):
            off = hw0 + ci * P
            pltpu.make_async_copy(f_hbm.at[bidx, :, pl.ds(off, P)], f_buf.at[slot], sems.at[slot]).wait()
            pltpu.make_async_copy(fo_hbm.at[bidx, :, pl.ds(off, P)], fo_buf.at[slot], sems.at[slot]).wait()
            pltpu.make_async_copy(oo_hbm.at[bidx, :, pl.ds(off, P)], o_buf.at[slot], sems.at[slot]).wait()
            pltpu.make_async_copy(lab_hbm.at[bidx, pl.ds(off, P)], lab_buf.at[slot], sems.at[slot]).wait()

        for s in range(NBUF):
            start_copies(s, s)

        def compute(slot, ci):
            fb, fob, ob, lb = f_buf.at[slot], fo_buf.at[slot], o_buf.at[slot], lab_buf.at[slot]
            sls = [pl.ds(pg * L, L) for pg in range(PG)]
            ms = [ob[0, sls[pg]] for pg in range(PG)]
            ks = [jnp.zeros((L,), jnp.int32) for _ in range(PG)]
            for ch in range(1, CO):
                chv = jnp.full((L,), ch, jnp.int32)
                for pg in range(PG):
                    v = ob[ch, sls[pg]]
                    upd = v > ms[pg]
                    ms[pg] = jnp.where(upd, v, ms[pg])
                    ks[pg] = jnp.where(upd, chv, ks[pg])
            for pg in range(PG):
                ks[pg] = jnp.where(lb[sls[pg]] < noc, ks[pg], 0)
                plsc.addupdate_scatter(n_acc, [ks[pg] * NREP + lane_rep], ones)
                ks[pg] = ks[pg] * NREP + lane_rep

            q0 = tuple(jnp.zeros((L,), jnp.float32) for _ in range(PG))

            @plsc.parallel_loop(0, C, step=CUNROLL, unroll=4, carry=q0)
            def qs(cc, qcarry):
                out = list(qcarry)
                for u in range(CUNROLL):
                    c = cc + u
                    base = c * (CO * NREP)
                    for pg in range(PG):
                        sl = pl.ds(pg * L, L)
                        d = fb[c, sl] - fob[c, sl]
                        plsc.addupdate_scatter(s_acc, [ks[pg] + base], d)
                        out[pg] = out[pg] + d * d
                return tuple(out)
            for pg in range(PG):
                plsc.addupdate_scatter(q_acc, [ks[pg]], qs[pg])

        def outer(g, carry):
            base = g * NBUF
            for s in range(NBUF):
                ci = base + s
                wait_copies(s, ci)
                compute(s, ci)

                @pl.when(ci + NBUF < NCHUNK)
                def _():
                    start_copies(s, ci + NBUF)
            return carry
        lax.fori_loop(0, NCHUNK // NBUF, outer, 0)

        base16 = lax.iota(jnp.int32, L) * NREP

        @plsc.parallel_loop(0, SFLAT // L, step=1, unroll=2)
        def _(i):
            idx0 = i * (L * NREP) + base16
            acc = plsc.load_gather(s_acc, [idx0])
            for r in range(1, NREP):
                acc = acc + plsc.load_gather(s_acc, [idx0 + r])
            s_red[pl.ds(i * L, L)] = acc

        def redqn(i, carry):
            idx0 = i * (L * NREP) + base16
            qa = plsc.load_gather(q_acc, [idx0])
            na = plsc.load_gather(n_acc, [idx0])
            for r in range(1, NREP):
                qa = qa + plsc.load_gather(q_acc, [idx0 + r])
                na = na + plsc.load_gather(n_acc, [idx0 + r])
            q_red[pl.ds(i * L, L)] = qa
            n_red[pl.ds(i * L, L)] = na
            return carry
        lax.fori_loop(0, 32 // L, redqn, 0)

        pltpu.sync_copy(s_red, s_out.at[wid])
        pltpu.sync_copy(q_red, q_out.at[wid])
        pltpu.sync_copy(n_red, n_out.at[wid])

    return body(f, fo, oo, lab, noc_vec)


def _tc_partials_body(noc_ref, f_ref, fo_ref, oo_ref, lab_ref,
                      s_ref, q_ref, n_ref, s_scr, q_scr, n_scr):
    step = pl.program_id(0)

    @pl.when(step == 0)
    def _():
        s_scr[...] = jnp.zeros_like(s_scr)
        q_scr[...] = jnp.zeros_like(q_scr)
        n_scr[...] = jnp.zeros_like(n_scr)

    d = f_ref[0] - fo_ref[0]
    oo = oo_ref[0]
    m = jnp.max(oo, axis=0, keepdims=True)
    chi = lax.broadcasted_iota(jnp.int32, (CO, PB), 0)
    idx = jnp.min(jnp.where(oo == m, chi, CO), axis=0, keepdims=True)
    lab = lab_ref[0]
    idx = jnp.where(lab < noc_ref[0], idx, 0)
    onehot_t = (chi == idx).astype(jnp.float32)

    s_scr[...] += lax.dot_general(d, onehot_t, (((1,), (1,)), ((), ())),
                                  preferred_element_type=jnp.float32)
    rowsq = jnp.sum(d * d, axis=0, keepdims=True)
    q_scr[...] += lax.dot_general(rowsq, onehot_t, (((1,), (1,)), ((), ())),
                                  preferred_element_type=jnp.float32)
    ones_row = jnp.ones((1, PB), jnp.float32)
    n_scr[...] += lax.dot_general(ones_row, onehot_t, (((1,), (1,)), ((), ())),
                                  preferred_element_type=jnp.float32)

    @pl.when(step == pl.num_programs(0) - 1)
    def _():
        s_ref[...] = s_scr[...]
        q_ref[...] = q_scr[...]
        n_ref[...] = n_scr[...]


def _tc_partials(f, fo, oo, lab3, noc11):
    nblk = (HW - X0) // PB
    steps = B * nblk

    def bmap(i):
        return i // nblk

    def pmap(i):
        return X0 // PB + i % nblk

    return pl.pallas_call(
        _tc_partials_body,
        grid=(steps,),
        in_specs=[
            pl.BlockSpec(memory_space=pltpu.SMEM),
            pl.BlockSpec((1, C, PB), lambda i: (bmap(i), 0, pmap(i))),
            pl.BlockSpec((1, C, PB), lambda i: (bmap(i), 0, pmap(i))),
            pl.BlockSpec((1, CO, PB), lambda i: (bmap(i), 0, pmap(i))),
            pl.BlockSpec((1, 1, PB), lambda i: (bmap(i), 0, pmap(i))),
        ],
        out_specs=[
            pl.BlockSpec((C, CO), lambda i: (0, 0)),
            pl.BlockSpec((1, CO), lambda i: (0, 0)),
            pl.BlockSpec((1, CO), lambda i: (0, 0)),
        ],
        out_shape=[
            jax.ShapeDtypeStruct((C, CO), jnp.float32),
            jax.ShapeDtypeStruct((1, CO), jnp.float32),
            jax.ShapeDtypeStruct((1, CO), jnp.float32),
        ],
        scratch_shapes=[
            pltpu.VMEM((C, CO), jnp.float32),
            pltpu.VMEM((1, CO), jnp.float32),
            pltpu.VMEM((1, CO), jnp.float32),
        ],
    )(noc11, f, fo, oo, lab3)


HAS_SC = X0 > 0
HAS_TC = X0 < HW


def _combine_body(*refs):
    i = 0
    st = jnp.zeros((C, CO), jnp.float32)
    q = jnp.zeros((1, CO), jnp.float32)
    n = jnp.zeros((1, CO), jnp.float32)
    if HAS_SC:
        s_sc, q_sc, n_sc = refs[0], refs[1], refs[2]
        i = 3
        st = st + jnp.sum(s_sc[...], axis=0)
        q = q + jnp.sum(q_sc[...], axis=0, keepdims=True)[:, :CO]
        n = n + jnp.sum(n_sc[...], axis=0, keepdims=True)[:, :CO]
    if HAS_TC:
        st = st + refs[i][...]
        q = q + refs[i + 1][...]
        n = n + refs[i + 2][...]
    o_ref = refs[-1]
    ss = jnp.sum(st * st, axis=0, keepdims=True)
    cls = lax.broadcasted_iota(jnp.int32, (1, CO), 1)
    denom = jnp.maximum(n, 1.0)
    loss_cl = q / denom - ss / (denom * denom)
    valid = (cls >= 1) & (n > 0.0)
    total = jnp.sum(jnp.where(valid, loss_cl, 0.0))
    present = jnp.sum(jnp.where(valid, 1.0, 0.0))
    loss = jnp.where(present > 0.0, total / jnp.maximum(present, 1.0), 0.0)
    o_ref[...] = jnp.reshape(loss, (1, 1))


def kernel(features, features_old, outputs_old, labels, prototypes, num_old_class):
    del prototypes
    f = features.reshape(B, C, HW)
    fo = features_old.reshape(B, C, HW)
    oo = outputs_old.reshape(B, CO, HW)
    lab = labels.reshape(B, HW)

    operands = []
    if HAS_SC:
        noc_vec = jnp.full((L,), num_old_class, jnp.int32)
        s_sc, q_sc, n_sc = _sc_partials(f, fo, oo, lab, noc_vec)
        operands += [s_sc.reshape(NW, C, CO), q_sc, n_sc]
    if HAS_TC:
        noc11 = jnp.asarray(num_old_class, jnp.int32).reshape(1)
        s_tc, q_tc, n_tc = _tc_partials(f, fo, oo, lab.reshape(B, 1, HW), noc11)
        operands += [s_tc, q_tc, n_tc]

    out = pl.pallas_call(
        _combine_body,
        out_shape=jax.ShapeDtypeStruct((1, 1), jnp.float32),
    )(*operands)
    return out[0, 0]


# speedup vs baseline: 1.4501x; 1.0003x over previous
import functools

import jax
import jax.numpy as jnp
from jax import lax
from jax.experimental import pallas as pl
from jax.experimental.pallas import tpu as pltpu
from jax.experimental.pallas import tpu_sc as plsc

NC, NS, L = 2, 16, 16
NW = NC * NS
B, C, H, W = 8, 256, 128, 128
HW = H * W
CO = 21
CHUNKS_PER_B = NW // B

P = 64
PG = P // L
NBUF = 2
PIX_PER_W = HW // CHUNKS_PER_B
NCHUNK = PIX_PER_W // P
CUNROLL = 2
SFLAT = C * CO
NREP = 8


def _sc_partials(f, fo, oo, lab, noc_vec):
    mesh = plsc.VectorSubcoreMesh(core_axis_name="c", subcore_axis_name="s")

    @functools.partial(
        pl.kernel,
        out_type=(
            jax.ShapeDtypeStruct((NW, SFLAT), jnp.float32),
            jax.ShapeDtypeStruct((NW, 32), jnp.float32),
            jax.ShapeDtypeStruct((NW, 32), jnp.float32),
        ),
        mesh=mesh,
        compiler_params=pltpu.CompilerParams(
            needs_layout_passes=False, use_tc_tiling_on_sc=False),
        scratch_types=[
            pltpu.VMEM((NBUF, C, P), jnp.float32),
            pltpu.VMEM((NBUF, C, P), jnp.float32),
            pltpu.VMEM((NBUF, CO, P), jnp.float32),
            pltpu.VMEM((NBUF, P), jnp.int32),
            pltpu.VMEM((L,), jnp.int32),
            pltpu.VMEM((NREP * SFLAT,), jnp.float32),
            pltpu.VMEM((NREP * 32,), jnp.float32),
            pltpu.VMEM((NREP * 32,), jnp.float32),
            pltpu.VMEM((SFLAT,), jnp.float32),
            pltpu.VMEM((32,), jnp.float32),
            pltpu.VMEM((32,), jnp.float32),
            pltpu.SemaphoreType.DMA((NBUF,)),
        ],
    )
    def body(f_hbm, fo_hbm, oo_hbm, lab_hbm, noc_hbm,
             s_out, q_out, n_out,
             f_buf, fo_buf, o_buf, lab_buf, noc_buf, s_acc, q_acc, n_acc,
             s_red, q_red, n_red, sems):
        cid = lax.axis_index("c")
        sid = lax.axis_index("s")
        wid = sid * NC + cid
        bidx = wid // CHUNKS_PER_B
        hw0 = (wid % CHUNKS_PER_B) * PIX_PER_W

        zero = jnp.zeros((L,), jnp.float32)

        @plsc.parallel_loop(0, NREP * SFLAT // L, step=1, unroll=8)
        def _(i):
            s_acc[pl.ds(i * L, L)] = zero

        @plsc.parallel_loop(0, NREP * 32 // L, step=1, unroll=2)
        def _(i):
            q_acc[pl.ds(i * L, L)] = zero
            n_acc[pl.ds(i * L, L)] = zero

        pltpu.sync_copy(noc_hbm, noc_buf)
        noc = noc_buf[...]
        ones = jnp.full((L,), 1.0, jnp.float32)
        lane_rep = lax.iota(jnp.int32, L) % NREP

        def start_copies(slot, ci):
            off = hw0 + ci * P
            pltpu.async_copy(f_hbm.at[bidx, :, pl.ds(off, P)], f_buf.at[slot], sems.at[slot])
            pltpu.async_copy(fo_hbm.at[bidx, :, pl.ds(off, P)], fo_buf.at[slot], sems.at[slot])
            pltpu.async_copy(oo_hbm.at[bidx, :, pl.ds(off, P)], o_buf.at[slot], sems.at[slot])
            pltpu.async_copy(lab_hbm.at[bidx, pl.ds(off, P)], lab_buf.at[slot], sems.at[slot])

        def wait_copies(slot, ci):
            off = hw0 + ci * P
            pltpu.make_async_copy(f_hbm.at[bidx, :, pl.ds(off, P)], f_buf.at[slot], sems.at[slot]).wait()
            pltpu.make_async_copy(fo_hbm.at[bidx, :, pl.ds(off, P)], fo_buf.at[slot], sems.at[slot]).wait()
            pltpu.make_async_copy(oo_hbm.at[bidx, :, pl.ds(off, P)], o_buf.at[slot], sems.at[slot]).wait()
            pltpu.make_async_copy(lab_hbm.at[bidx, pl.ds(off, P)], lab_buf.at[slot], sems.at[slot]).wait()

        for s in range(NBUF):
            start_copies(s, s)

        def compute(slot, ci):
            fb, fob, ob, lb = f_buf.at[slot], fo_buf.at[slot], o_buf.at[slot], lab_buf.at[slot]
            sls = [pl.ds(pg * L, L) for pg in range(PG)]
            ms = [ob[0, sls[pg]] for pg in range(PG)]
            ks = [jnp.zeros((L,), jnp.int32) for _ in range(PG)]
            for ch in range(1, CO):
                chv = jnp.full((L,), ch, jnp.int32)
                for pg in range(PG):
                    v = ob[ch, sls[pg]]
                    upd = v > ms[pg]
                    ms[pg] = jnp.where(upd, v, ms[pg])
                    ks[pg] = jnp.where(upd, chv, ks[pg])
            for pg in range(PG):
                ks[pg] = jnp.where(lb[sls[pg]] < noc, ks[pg], 0)
                plsc.addupdate_scatter(n_acc, [ks[pg] * NREP + lane_rep], ones)
                ks[pg] = ks[pg] * NREP + lane_rep

            q0 = tuple(jnp.zeros((L,), jnp.float32) for _ in range(PG))

            @plsc.parallel_loop(0, C, step=CUNROLL, unroll=4, carry=q0)
            def qs(cc, qcarry):
                out = list(qcarry)
                for u in range(CUNROLL):
                    c = cc + u
                    base = c * (CO * NREP)
                    for pg in range(PG):
                        sl = pl.ds(pg * L, L)
                        d = fb[c, sl] - fob[c, sl]
                        plsc.addupdate_scatter(s_acc, [ks[pg] + base], d)
                        out[pg] = out[pg] + d * d
                return tuple(out)
            for pg in range(PG):
                plsc.addupdate_scatter(q_acc, [ks[pg]], qs[pg])

        def outer(g, carry):
            base = g * NBUF
            for s in range(NBUF):
                ci = base + s
                wait_copies(s, ci)
                compute(s, ci)

                @pl.when(ci + NBUF < NCHUNK)
                def _():
                    start_copies(s, ci + NBUF)
            return carry
        lax.fori_loop(0, NCHUNK // NBUF, outer, 0)

        base16 = lax.iota(jnp.int32, L) * NREP

        @plsc.parallel_loop(0, SFLAT // L, step=1, unroll=2)
        def _(i):
            idx0 = i * (L * NREP) + base16
            acc = plsc.load_gather(s_acc, [idx0])
            for r in range(1, NREP):
                acc = acc + plsc.load_gather(s_acc, [idx0 + r])
            s_red[pl.ds(i * L, L)] = acc

        def redqn(i, carry):
            idx0 = i * (L * NREP) + base16
            qa = plsc.load_gather(q_acc, [idx0])
            na = plsc.load_gather(n_acc, [idx0])
            for r in range(1, NREP):
                qa = qa + plsc.load_gather(q_acc, [idx0 + r])
                na = na + plsc.load_gather(n_acc, [idx0 + r])
            q_red[pl.ds(i * L, L)] = qa
            n_red[pl.ds(i * L, L)] = na
            return carry
        lax.fori_loop(0, 32 // L, redqn, 0)

        pltpu.sync_copy(s_red, s_out.at[wid])
        pltpu.sync_copy(q_red, q_out.at[wid])
        pltpu.sync_copy(n_red, n_out.at[wid])

    return body(f, fo, oo, lab, noc_vec)


def _combine_body(s_sc, q_sc, n_sc, o_ref):
    st = jnp.sum(s_sc[...], axis=0)
    q = jnp.sum(q_sc[...], axis=0, keepdims=True)[:, :CO]
    n = jnp.sum(n_sc[...], axis=0, keepdims=True)[:, :CO]
    ss = jnp.sum(st * st, axis=0, keepdims=True)
    cls = lax.broadcasted_iota(jnp.int32, (1, CO), 1)
    denom = jnp.maximum(n, 1.0)
    loss_cl = q / denom - ss / (denom * denom)
    valid = (cls >= 1) & (n > 0.0)
    total = jnp.sum(jnp.where(valid, loss_cl, 0.0))
    present = jnp.sum(jnp.where(valid, 1.0, 0.0))
    loss = jnp.where(present > 0.0, total / jnp.maximum(present, 1.0), 0.0)
    o_ref[...] = jnp.reshape(loss, (1, 1))


def kernel(features, features_old, outputs_old, labels, prototypes, num_old_class):
    del prototypes
    f = features.reshape(B, C, HW)
    fo = features_old.reshape(B, C, HW)
    oo = outputs_old.reshape(B, CO, HW)
    lab = labels.reshape(B, HW)

    noc_vec = jnp.full((L,), num_old_class, jnp.int32)
    s_sc, q_sc, n_sc = _sc_partials(f, fo, oo, lab, noc_vec)

    out = pl.pallas_call(
        _combine_body,
        out_shape=jax.ShapeDtypeStruct((1, 1), jnp.float32),
    )(s_sc.reshape(NW, C, CO), q_sc, n_sc)
    return out[0, 0]
